# baseline (device time: 216901 ns/iter reference)
import jax
import jax.numpy as jnp
from jax import lax
from jax.experimental import pallas as pl
from jax.experimental.pallas import tpu as pltpu

N_DEV = 4
SQ = 2048
D_MODEL = 1024
HEADS_PER_SHARD = 8
DH = 128
LOG2E = 1.4426950408889634
SCALE = 0.08838834764831843 * LOG2E

BLK = 64
NQB = 11
ROWS = NQB * BLK
RTOT = 3 * ROWS

CH = ROWS // N_DEV
HF = CH // 2

N_HOPS = 2 * (N_DEV - 1)
GRID = 3 * HEADS_PER_SHARD + N_HOPS + 2

NEG = -1e9


def _class_rows(a3):
    d = a3.shape[-1]
    return jnp.concatenate(
        [a3[0::3], a3[1::3], a3[2::3], a3[0:1]], axis=0
    ).reshape(RTOT, d)


def _prep_body(k2_ref, v2_ref, kg_ref, vg_ref):
    r = pl.program_id(1)
    for i in range(NQB - 1):
        src = pl.ds((3 * i + r) * BLK, BLK)
        dst = pl.ds(i * BLK, BLK)
        kg_ref[0, 0, dst, :] = k2_ref[src, :].astype(jnp.bfloat16)
        vg_ref[0, 0, dst, :] = v2_ref[src, :].astype(jnp.bfloat16)
    src = pl.ds(jnp.where(r == 2, 0, (30 + r) * BLK), BLK)
    dst = pl.ds((NQB - 1) * BLK, BLK)
    kg_ref[0, 0, dst, :] = k2_ref[src, :].astype(jnp.bfloat16)
    vg_ref[0, 0, dst, :] = v2_ref[src, :].astype(jnp.bfloat16)


def _prep_kv(K2, V2):
    return pl.pallas_call(
        _prep_body,
        grid=(HEADS_PER_SHARD, 3),
        in_specs=[
            pl.BlockSpec((SQ, DH), lambda h, r: (0, lax.axis_index("i") * HEADS_PER_SHARD + h)),
            pl.BlockSpec((SQ, DH), lambda h, r: (0, lax.axis_index("i") * HEADS_PER_SHARD + h)),
        ],
        out_specs=[
            pl.BlockSpec((1, 1, ROWS, DH), lambda h, r: (h, r, 0, 0)),
            pl.BlockSpec((1, 1, ROWS, DH), lambda h, r: (h, r, 0, 0)),
        ],
        out_shape=[
            jax.ShapeDtypeStruct((HEADS_PER_SHARD, 3, ROWS, DH), jnp.bfloat16),
            jax.ShapeDtypeStruct((HEADS_PER_SHARD, 3, ROWS, DH), jnp.bfloat16),
        ],
    )(K2, V2)


def _compute_step(g, xp_ref, wq_ref, ksel_ref, vsel_ref, kdiag_ref, vdiag_ref,
                  k0_ref, v0_ref, wo_ref, out_ref):
    c = g // HEADS_PER_SHARD
    h = lax.rem(g, HEADS_PER_SHARD)

    q = jax.lax.dot(
        xp_ref[...], wq_ref[...], preferred_element_type=jnp.float32
    ) * SCALE
    qb16 = q.astype(jnp.bfloat16)

    s1 = lax.dot_general(
        qb16, ksel_ref[0, 0], (((1,), (1,)), ((), ())),
        preferred_element_type=jnp.float32,
    )
    w1 = jnp.exp2(s1.astype(jnp.bfloat16))

    s0 = lax.dot_general(
        qb16, k0_ref[0, 0, 0], (((1,), (1,)), ((), ())),
        preferred_element_type=jnp.float32,
    )
    w0 = jnp.exp2((s0 + jnp.where(c == 2, 0.0, NEG)).astype(jnp.bfloat16))

    q3 = qb16.reshape(NQB, BLK, DH)
    kd3 = kdiag_ref[0, 0].reshape(NQB, BLK, DH)
    s2 = lax.dot_general(
        q3, kd3, (((2,), (2,)), ((0,), (0,))),
        preferred_element_type=jnp.float32,
    )
    w2 = jnp.exp2((s2 + jnp.where(c == 0, NEG, 0.0)).astype(jnp.bfloat16))

    denom = (
        jnp.sum(w1, axis=1, keepdims=True, dtype=jnp.float32).reshape(NQB, BLK, 1)
        + jnp.sum(w0, axis=1, keepdims=True, dtype=jnp.float32).reshape(NQB, BLK, 1)
        + jnp.sum(w2, axis=2, keepdims=True, dtype=jnp.float32)
    )

    ctx = (
        lax.dot_general(
            w1, vsel_ref[0, 0], (((1,), (0,)), ((), ())),
            preferred_element_type=jnp.float32,
        )
        + lax.dot_general(
            w0, v0_ref[0, 0, 0], (((1,), (0,)), ((), ())),
            preferred_element_type=jnp.float32,
        )
    ).reshape(NQB, BLK, DH)
    vd3 = vdiag_ref[0, 0].reshape(NQB, BLK, DH)
    ctx = ctx + lax.dot_general(
        w2, vd3, (((2,), (1,)), ((0,), (0,))),
        preferred_element_type=jnp.float32,
    )
    ctx = ctx / denom

    o = lax.dot_general(
        ctx.astype(jnp.bfloat16).reshape(ROWS, DH), wo_ref[0],
        (((1,), (0,)), ((), ())),
        preferred_element_type=jnp.float32,
    )

    row0 = c * ROWS

    @pl.when(h == 0)
    def _():
        out_ref[pl.ds(row0, ROWS), :] = o

    @pl.when(h != 0)
    def _():
        out_ref[pl.ds(row0, ROWS), :] += o


def _fused_body(xp_ref, wq_ref, ksel_ref, vsel_ref, kdiag_ref, vdiag_ref,
                k0_ref, v0_ref, wo_ref, final_ref,
                out_ref, sstage_ref, comm_ref, send_sems, recv_sems):
    g = pl.program_id(0)
    p = lax.axis_index("i")
    left = (p - 1 + N_DEV) % N_DEV
    right = (p + 1) % N_DEV

    @pl.when(g == 0)
    def _():
        barrier_sem = pltpu.get_barrier_semaphore()
        for nbr in (left, right):
            pl.semaphore_signal(
                barrier_sem, inc=1,
                device_id=(nbr,), device_id_type=pl.DeviceIdType.MESH,
            )
        pl.semaphore_wait(barrier_sem, 2)

    @pl.when(g < 3 * HEADS_PER_SHARD)
    def _():
        _compute_step(g, xp_ref, wq_ref, ksel_ref, vsel_ref, kdiag_ref,
                      vdiag_ref, k0_ref, v0_ref, wo_ref, out_ref)

    cs = g // HEADS_PER_SHARD - 1
    k = g - HEADS_PER_SHARD * (cs + 1)
    base = cs * ROWS

    def _desc(d, hop, src_ref, dst_ref, target):
        return pltpu.make_async_remote_copy(
            src_ref=src_ref, dst_ref=dst_ref,
            send_sem=send_sems.at[d, cs, hop], recv_sem=recv_sems.at[d, cs, hop],
            device_id=(target,), device_id_type=pl.DeviceIdType.MESH,
        )

    @pl.when((g >= HEADS_PER_SHARD) & (k >= 1) & (k <= N_HOPS))
    def _():
        ka = k - 1
        for d, tgt in ((0, right), (1, left)):
            desc = _desc(d, ka, sstage_ref.at[d, cs, 0], comm_ref.at[d, cs, ka], tgt)
            desc.wait_send()
            desc.wait_recv()

        @pl.when(ka <= N_DEV - 2)
        def _():
            j_cw = (p - ka - 1 + N_DEV) % N_DEV
            j_ccw = (p + ka + 1) % N_DEV
            out_ref[pl.ds(base + j_cw * CH, HF), :] += comm_ref[0, cs, ka].astype(jnp.float32)
            out_ref[pl.ds(base + j_ccw * CH + HF, HF), :] += comm_ref[1, cs, ka].astype(jnp.float32)

        @pl.when(ka > N_DEV - 2)
        def _():
            ta = ka - (N_DEV - 1)
            j_cw = (p - ta + N_DEV) % N_DEV
            j_ccw = (p + ta) % N_DEV
            out_ref[pl.ds(base + j_cw * CH, HF), :] = comm_ref[0, cs, ka].astype(jnp.float32)
            out_ref[pl.ds(base + j_ccw * CH + HF, HF), :] = comm_ref[1, cs, ka].astype(jnp.float32)

    @pl.when((g >= HEADS_PER_SHARD) & (k <= N_HOPS - 1))
    def _():
        @pl.when(k <= N_DEV - 2)
        def _():
            i_cw = (p - k + N_DEV) % N_DEV
            i_ccw = (p + k) % N_DEV
            sstage_ref[0, cs, k] = out_ref[pl.ds(base + i_cw * CH, HF), :].astype(jnp.bfloat16)
            sstage_ref[1, cs, k] = out_ref[pl.ds(base + i_ccw * CH + HF, HF), :].astype(jnp.bfloat16)
            for d, tgt in ((0, right), (1, left)):
                _desc(d, k, sstage_ref.at[d, cs, k], comm_ref.at[d, cs, k], tgt).start()

        @pl.when(k == N_DEV - 1)
        def _():
            own_cw = (p + 1) % N_DEV
            own_ccw = (p - 1 + N_DEV) % N_DEV
            sstage_ref[0, cs, k] = out_ref[pl.ds(base + own_cw * CH, HF), :].astype(jnp.bfloat16)
            sstage_ref[1, cs, k] = out_ref[pl.ds(base + own_ccw * CH + HF, HF), :].astype(jnp.bfloat16)
            for d, tgt in ((0, right), (1, left)):
                _desc(d, k, sstage_ref.at[d, cs, k], comm_ref.at[d, cs, k], tgt).start()

        @pl.when(k > N_DEV - 1)
        def _():
            for d, tgt in ((0, right), (1, left)):
                _desc(d, k, comm_ref.at[d, cs, k - 1], comm_ref.at[d, cs, k], tgt).start()

    first_copy = 2 * HEADS_PER_SHARD - 1
    @pl.when((g >= first_copy) & (lax.rem(g - first_copy, HEADS_PER_SHARD) == 0))
    def _():
        cc = (g - first_copy) // HEADS_PER_SHARD
        for i in range(NQB - 1):
            final_ref[pl.ds((3 * i + cc) * BLK, BLK), :] = (
                out_ref[pl.ds((cc * NQB + i) * BLK, BLK), :])

        @pl.when(cc <= 1)
        def _():
            final_ref[pl.ds((30 + cc) * BLK, BLK), :] = (
                out_ref[pl.ds((cc * NQB + NQB - 1) * BLK, BLK), :])


def _clamp_c(g):
    return jnp.minimum(g // HEADS_PER_SHARD, 2)


def _fused(xp, wqb, kg, vg, wob):
    kg5 = kg.reshape(HEADS_PER_SHARD, 3, NQB, BLK, DH)
    vg5 = vg.reshape(HEADS_PER_SHARD, 3, NQB, BLK, DH)
    h_of = lambda g: lax.rem(g, HEADS_PER_SHARD)
    return pl.pallas_call(
        _fused_body,
        grid=(GRID,),
        in_specs=[
            pl.BlockSpec((ROWS, D_MODEL), lambda g: (_clamp_c(g), 0)),
            pl.BlockSpec((D_MODEL, DH), lambda g: (0, h_of(g))),
            pl.BlockSpec((1, 1, ROWS, DH), lambda g: (h_of(g), (3 - _clamp_c(g)) % 3, 0, 0)),
            pl.BlockSpec((1, 1, ROWS, DH), lambda g: (h_of(g), (3 - _clamp_c(g)) % 3, 0, 0)),
            pl.BlockSpec((1, 1, ROWS, DH), lambda g: (h_of(g), _clamp_c(g), 0, 0)),
            pl.BlockSpec((1, 1, ROWS, DH), lambda g: (h_of(g), _clamp_c(g), 0, 0)),
            pl.BlockSpec((1, 1, 1, BLK, DH), lambda g: (h_of(g), 0, 0, 0, 0)),
            pl.BlockSpec((1, 1, 1, BLK, DH), lambda g: (h_of(g), 0, 0, 0, 0)),
            pl.BlockSpec((1, DH, D_MODEL), lambda g: (h_of(g), 0, 0)),
        ],
        out_specs=pl.BlockSpec(memory_space=pltpu.VMEM),
        out_shape=jax.ShapeDtypeStruct((SQ, D_MODEL), jnp.float32),
        scratch_shapes=[
            pltpu.VMEM((RTOT, D_MODEL), jnp.float32),
            pltpu.VMEM((2, 3, N_HOPS, HF, D_MODEL), jnp.bfloat16),
            pltpu.VMEM((2, 3, N_HOPS, HF, D_MODEL), jnp.bfloat16),
            pltpu.SemaphoreType.DMA((2, 3, N_HOPS)),
            pltpu.SemaphoreType.DMA((2, 3, N_HOPS)),
        ],
        compiler_params=pltpu.CompilerParams(collective_id=0),
    )(xp, wqb, kg, vg, kg, vg, kg5, vg5, wob)


def kernel(x, Wq, K_ext, V_ext, Wo):
    p = lax.axis_index("i")

    xb = x[0].astype(jnp.bfloat16)
    xp = _class_rows(xb.reshape(32, BLK, D_MODEL))
    wqb = Wq.astype(jnp.bfloat16)

    kg, vg = _prep_kv(
        K_ext[0].reshape(SQ, 32 * DH), V_ext[0].reshape(SQ, 32 * DH)
    )

    wob = Wo.reshape(HEADS_PER_SHARD, DH, D_MODEL).astype(jnp.bfloat16)

    total = _fused(xp, wqb, kg, vg, wob)
    return total[None]


# device time: 139301 ns/iter; 1.5571x vs baseline; 1.5571x over previous
import jax
import jax.numpy as jnp
from jax import lax
from jax.experimental import pallas as pl
from jax.experimental.pallas import tpu as pltpu

N_DEV = 4
SQ = 2048
D_MODEL = 1024
HEADS_PER_SHARD = 8
DH = 128
LOG2E = 1.4426950408889634
SCALE = 0.08838834764831843 * LOG2E

BLK = 64
NQB = 11
ROWS = NQB * BLK
RTOT = 3 * ROWS

CH = ROWS // N_DEV
HF = CH // 2

N_HOPS = 2 * (N_DEV - 1)
GRID = 3 * HEADS_PER_SHARD + N_HOPS + 2

NEG = -1e9


def _class_rows(a3):
    d = a3.shape[-1]
    return jnp.concatenate(
        [a3[0::3], a3[1::3], a3[2::3], a3[0:1]], axis=0
    ).reshape(RTOT, d)


def _group_blocks(a4):
    return jnp.concatenate(
        [a4[:, 0::3], a4[:, 1::3], a4[:, 2::3], a4[:, 0:1]], axis=1
    ).reshape(HEADS_PER_SHARD, 3, ROWS, DH)


def _compute_step(g, xp_ref, wq_ref, ksel_ref, vsel_ref, kdiag_ref, vdiag_ref,
                  k0_ref, v0_ref, wo_ref, out_ref):
    c = g // HEADS_PER_SHARD
    h = lax.rem(g, HEADS_PER_SHARD)

    q = jax.lax.dot(
        xp_ref[...], wq_ref[...], preferred_element_type=jnp.float32
    ) * SCALE
    qb16 = q.astype(jnp.bfloat16)

    s1 = lax.dot_general(
        qb16, ksel_ref[0, 0], (((1,), (1,)), ((), ())),
        preferred_element_type=jnp.float32,
    )
    w1 = jnp.exp2(s1.astype(jnp.bfloat16))

    s0 = lax.dot_general(
        qb16, k0_ref[0, 0, 0], (((1,), (1,)), ((), ())),
        preferred_element_type=jnp.float32,
    )
    w0 = jnp.exp2((s0 + jnp.where(c == 2, 0.0, NEG)).astype(jnp.bfloat16))

    q3 = qb16.reshape(NQB, BLK, DH)
    kd3 = kdiag_ref[0, 0].reshape(NQB, BLK, DH)
    s2 = lax.dot_general(
        q3, kd3, (((2,), (2,)), ((0,), (0,))),
        preferred_element_type=jnp.float32,
    )
    w2 = jnp.exp2((s2 + jnp.where(c == 0, NEG, 0.0)).astype(jnp.bfloat16))

    denom = (
        jnp.sum(w1, axis=1, keepdims=True, dtype=jnp.float32).reshape(NQB, BLK, 1)
        + jnp.sum(w0, axis=1, keepdims=True, dtype=jnp.float32).reshape(NQB, BLK, 1)
        + jnp.sum(w2, axis=2, keepdims=True, dtype=jnp.float32)
    )

    ctx = (
        lax.dot_general(
            w1, vsel_ref[0, 0], (((1,), (0,)), ((), ())),
            preferred_element_type=jnp.float32,
        )
        + lax.dot_general(
            w0, v0_ref[0, 0, 0], (((1,), (0,)), ((), ())),
            preferred_element_type=jnp.float32,
        )
    ).reshape(NQB, BLK, DH)
    vd3 = vdiag_ref[0, 0].reshape(NQB, BLK, DH)
    ctx = ctx + lax.dot_general(
        w2, vd3, (((2,), (1,)), ((0,), (0,))),
        preferred_element_type=jnp.float32,
    )
    ctx = ctx / denom

    o = lax.dot_general(
        ctx.astype(jnp.bfloat16).reshape(ROWS, DH), wo_ref[0],
        (((1,), (0,)), ((), ())),
        preferred_element_type=jnp.float32,
    )

    row0 = c * ROWS

    @pl.when(h == 0)
    def _():
        out_ref[pl.ds(row0, ROWS), :] = o

    @pl.when(h != 0)
    def _():
        out_ref[pl.ds(row0, ROWS), :] += o


def _fused_body(xp_ref, wq_ref, ksel_ref, vsel_ref, kdiag_ref, vdiag_ref,
                k0_ref, v0_ref, wo_ref, final_ref,
                out_ref, sstage_ref, comm_ref, send_sems, recv_sems):
    g = pl.program_id(0)
    p = lax.axis_index("i")
    left = (p - 1 + N_DEV) % N_DEV
    right = (p + 1) % N_DEV

    @pl.when(g == 0)
    def _():
        barrier_sem = pltpu.get_barrier_semaphore()
        for nbr in (left, right):
            pl.semaphore_signal(
                barrier_sem, inc=1,
                device_id=(nbr,), device_id_type=pl.DeviceIdType.MESH,
            )
        pl.semaphore_wait(barrier_sem, 2)

    @pl.when(g < 3 * HEADS_PER_SHARD)
    def _():
        _compute_step(g, xp_ref, wq_ref, ksel_ref, vsel_ref, kdiag_ref,
                      vdiag_ref, k0_ref, v0_ref, wo_ref, out_ref)

    cs = g // HEADS_PER_SHARD - 1
    k = g - HEADS_PER_SHARD * (cs + 1)
    base = cs * ROWS

    def _desc(d, hop, src_ref, dst_ref, target):
        return pltpu.make_async_remote_copy(
            src_ref=src_ref, dst_ref=dst_ref,
            send_sem=send_sems.at[d, cs, hop], recv_sem=recv_sems.at[d, cs, hop],
            device_id=(target,), device_id_type=pl.DeviceIdType.MESH,
        )

    @pl.when((g >= HEADS_PER_SHARD) & (k >= 1) & (k <= N_HOPS))
    def _():
        ka = k - 1
        for d, tgt in ((0, right), (1, left)):
            desc = _desc(d, ka, sstage_ref.at[d, cs, 0], comm_ref.at[d, cs, ka], tgt)
            desc.wait_send()
            desc.wait_recv()

        @pl.when(ka <= N_DEV - 2)
        def _():
            j_cw = (p - ka - 1 + N_DEV) % N_DEV
            j_ccw = (p + ka + 1) % N_DEV
            out_ref[pl.ds(base + j_cw * CH, HF), :] += comm_ref[0, cs, ka].astype(jnp.float32)
            out_ref[pl.ds(base + j_ccw * CH + HF, HF), :] += comm_ref[1, cs, ka].astype(jnp.float32)

        @pl.when(ka > N_DEV - 2)
        def _():
            ta = ka - (N_DEV - 1)
            j_cw = (p - ta + N_DEV) % N_DEV
            j_ccw = (p + ta) % N_DEV
            out_ref[pl.ds(base + j_cw * CH, HF), :] = comm_ref[0, cs, ka].astype(jnp.float32)
            out_ref[pl.ds(base + j_ccw * CH + HF, HF), :] = comm_ref[1, cs, ka].astype(jnp.float32)

    @pl.when((g >= HEADS_PER_SHARD) & (k <= N_HOPS - 1))
    def _():
        @pl.when(k <= N_DEV - 2)
        def _():
            i_cw = (p - k + N_DEV) % N_DEV
            i_ccw = (p + k) % N_DEV
            sstage_ref[0, cs, k] = out_ref[pl.ds(base + i_cw * CH, HF), :].astype(jnp.bfloat16)
            sstage_ref[1, cs, k] = out_ref[pl.ds(base + i_ccw * CH + HF, HF), :].astype(jnp.bfloat16)
            for d, tgt in ((0, right), (1, left)):
                _desc(d, k, sstage_ref.at[d, cs, k], comm_ref.at[d, cs, k], tgt).start()

        @pl.when(k == N_DEV - 1)
        def _():
            own_cw = (p + 1) % N_DEV
            own_ccw = (p - 1 + N_DEV) % N_DEV
            sstage_ref[0, cs, k] = out_ref[pl.ds(base + own_cw * CH, HF), :].astype(jnp.bfloat16)
            sstage_ref[1, cs, k] = out_ref[pl.ds(base + own_ccw * CH + HF, HF), :].astype(jnp.bfloat16)
            for d, tgt in ((0, right), (1, left)):
                _desc(d, k, sstage_ref.at[d, cs, k], comm_ref.at[d, cs, k], tgt).start()

        @pl.when(k > N_DEV - 1)
        def _():
            for d, tgt in ((0, right), (1, left)):
                _desc(d, k, comm_ref.at[d, cs, k - 1], comm_ref.at[d, cs, k], tgt).start()

    first_copy = 2 * HEADS_PER_SHARD - 1
    @pl.when((g >= first_copy) & (lax.rem(g - first_copy, HEADS_PER_SHARD) == 0))
    def _():
        cc = (g - first_copy) // HEADS_PER_SHARD
        for i in range(NQB - 1):
            final_ref[pl.ds((3 * i + cc) * BLK, BLK), :] = (
                out_ref[pl.ds((cc * NQB + i) * BLK, BLK), :])

        @pl.when(cc <= 1)
        def _():
            final_ref[pl.ds((30 + cc) * BLK, BLK), :] = (
                out_ref[pl.ds((cc * NQB + NQB - 1) * BLK, BLK), :])


def _clamp_c(g):
    return jnp.minimum(g // HEADS_PER_SHARD, 2)


def _fused(xp, wqb, kg, vg, wob):
    kg5 = kg.reshape(HEADS_PER_SHARD, 3, NQB, BLK, DH)
    vg5 = vg.reshape(HEADS_PER_SHARD, 3, NQB, BLK, DH)
    h_of = lambda g: lax.rem(g, HEADS_PER_SHARD)
    return pl.pallas_call(
        _fused_body,
        grid=(GRID,),
        in_specs=[
            pl.BlockSpec((ROWS, D_MODEL), lambda g: (_clamp_c(g), 0)),
            pl.BlockSpec((D_MODEL, DH), lambda g: (0, h_of(g))),
            pl.BlockSpec((1, 1, ROWS, DH), lambda g: (h_of(g), (3 - _clamp_c(g)) % 3, 0, 0)),
            pl.BlockSpec((1, 1, ROWS, DH), lambda g: (h_of(g), (3 - _clamp_c(g)) % 3, 0, 0)),
            pl.BlockSpec((1, 1, ROWS, DH), lambda g: (h_of(g), _clamp_c(g), 0, 0)),
            pl.BlockSpec((1, 1, ROWS, DH), lambda g: (h_of(g), _clamp_c(g), 0, 0)),
            pl.BlockSpec((1, 1, 1, BLK, DH), lambda g: (h_of(g), 0, 0, 0, 0)),
            pl.BlockSpec((1, 1, 1, BLK, DH), lambda g: (h_of(g), 0, 0, 0, 0)),
            pl.BlockSpec((1, DH, D_MODEL), lambda g: (h_of(g), 0, 0)),
        ],
        out_specs=pl.BlockSpec(memory_space=pltpu.VMEM),
        out_shape=jax.ShapeDtypeStruct((SQ, D_MODEL), jnp.float32),
        scratch_shapes=[
            pltpu.VMEM((RTOT, D_MODEL), jnp.float32),
            pltpu.VMEM((2, 3, N_HOPS, HF, D_MODEL), jnp.bfloat16),
            pltpu.VMEM((2, 3, N_HOPS, HF, D_MODEL), jnp.bfloat16),
            pltpu.SemaphoreType.DMA((2, 3, N_HOPS)),
            pltpu.SemaphoreType.DMA((2, 3, N_HOPS)),
        ],
        compiler_params=pltpu.CompilerParams(collective_id=0),
    )(xp, wqb, kg, vg, kg, vg, kg5, vg5, wob)


def kernel(x, Wq, K_ext, V_ext, Wo):
    p = lax.axis_index("i")

    xb = x[0].astype(jnp.bfloat16)
    xp = _class_rows(xb.reshape(32, BLK, D_MODEL))
    wqb = Wq.astype(jnp.bfloat16)

    k = lax.dynamic_slice_in_dim(K_ext[0], p * HEADS_PER_SHARD, HEADS_PER_SHARD, axis=1)
    v = lax.dynamic_slice_in_dim(V_ext[0], p * HEADS_PER_SHARD, HEADS_PER_SHARD, axis=1)
    kb = jnp.transpose(k, (1, 0, 2)).astype(jnp.bfloat16)
    vb = jnp.transpose(v, (1, 0, 2)).astype(jnp.bfloat16)
    kg = _group_blocks(kb.reshape(HEADS_PER_SHARD, 32, BLK, DH))
    vg = _group_blocks(vb.reshape(HEADS_PER_SHARD, 32, BLK, DH))

    wob = Wo.reshape(HEADS_PER_SHARD, DH, D_MODEL).astype(jnp.bfloat16)

    total = _fused(xp, wqb, kg, vg, wob)
    return total[None]


# device time: 132463 ns/iter; 1.6374x vs baseline; 1.0516x over previous
import jax
import jax.numpy as jnp
from jax import lax
from jax.experimental import pallas as pl
from jax.experimental.pallas import tpu as pltpu

N_DEV = 4
SQ = 2048
D_MODEL = 1024
HEADS_PER_SHARD = 8
DH = 128
LOG2E = 1.4426950408889634
SCALE = 0.08838834764831843 * LOG2E

BLK = 64
NQB = 11
ROWS = NQB * BLK
RTOT = 3 * ROWS

CH = ROWS // N_DEV
HF = CH // 2

N_HOPS = 2 * (N_DEV - 1)
GRID = 3 * HEADS_PER_SHARD + N_HOPS + 2

NEG = -1e9


def _class_rows(a3):
    d = a3.shape[-1]
    return jnp.concatenate(
        [a3[0::3], a3[1::3], a3[2::3], a3[0:1]], axis=0
    ).reshape(RTOT, d)


def _group_blocks(a4):
    return jnp.concatenate(
        [a4[:, 0::3], a4[:, 1::3], a4[:, 2::3], a4[:, 0:1]], axis=1
    ).reshape(HEADS_PER_SHARD, 3, ROWS, DH)


def _compute_step(g, x_ref, wq_ref, ksel_ref, vsel_ref, kdiag_ref, vdiag_ref,
                  k0_ref, v0_ref, wo_ref, out_ref, xpb_ref):
    c = g // HEADS_PER_SHARD
    h = lax.rem(g, HEADS_PER_SHARD)

    @pl.when(h == 0)
    def _():
        for i in range(NQB - 1):
            xpb_ref[pl.ds(i * BLK, BLK), :] = (
                x_ref[pl.ds((3 * i + c) * BLK, BLK), :].astype(jnp.bfloat16))
        src = pl.ds(jnp.where(c == 2, 0, (30 + c) * BLK), BLK)
        xpb_ref[pl.ds((NQB - 1) * BLK, BLK), :] = x_ref[src, :].astype(jnp.bfloat16)

    q = jax.lax.dot(
        xpb_ref[...], wq_ref[...], preferred_element_type=jnp.float32
    ) * SCALE
    qb16 = q.astype(jnp.bfloat16)

    s1 = lax.dot_general(
        qb16, ksel_ref[0, 0], (((1,), (1,)), ((), ())),
        preferred_element_type=jnp.float32,
    )
    w1 = jnp.exp2(s1.astype(jnp.bfloat16))

    s0 = lax.dot_general(
        qb16, k0_ref[0, 0, 0], (((1,), (1,)), ((), ())),
        preferred_element_type=jnp.float32,
    )
    w0 = jnp.exp2((s0 + jnp.where(c == 2, 0.0, NEG)).astype(jnp.bfloat16))

    q3 = qb16.reshape(NQB, BLK, DH)
    kd3 = kdiag_ref[0, 0].reshape(NQB, BLK, DH)
    s2 = lax.dot_general(
        q3, kd3, (((2,), (2,)), ((0,), (0,))),
        preferred_element_type=jnp.float32,
    )
    w2 = jnp.exp2((s2 + jnp.where(c == 0, NEG, 0.0)).astype(jnp.bfloat16))

    denom = (
        jnp.sum(w1, axis=1, keepdims=True, dtype=jnp.float32).reshape(NQB, BLK, 1)
        + jnp.sum(w0, axis=1, keepdims=True, dtype=jnp.float32).reshape(NQB, BLK, 1)
        + jnp.sum(w2, axis=2, keepdims=True, dtype=jnp.float32)
    )

    ctx = (
        lax.dot_general(
            w1, vsel_ref[0, 0], (((1,), (0,)), ((), ())),
            preferred_element_type=jnp.float32,
        )
        + lax.dot_general(
            w0, v0_ref[0, 0, 0], (((1,), (0,)), ((), ())),
            preferred_element_type=jnp.float32,
        )
    ).reshape(NQB, BLK, DH)
    vd3 = vdiag_ref[0, 0].reshape(NQB, BLK, DH)
    ctx = ctx + lax.dot_general(
        w2, vd3, (((2,), (1,)), ((0,), (0,))),
        preferred_element_type=jnp.float32,
    )
    ctx = ctx / denom

    o = lax.dot_general(
        ctx.astype(jnp.bfloat16).reshape(ROWS, DH), wo_ref[0],
        (((1,), (0,)), ((), ())),
        preferred_element_type=jnp.float32,
    )

    row0 = c * ROWS

    @pl.when(h == 0)
    def _():
        out_ref[pl.ds(row0, ROWS), :] = o

    @pl.when(h != 0)
    def _():
        out_ref[pl.ds(row0, ROWS), :] += o


def _fused_body(x_ref, wq_ref, ksel_ref, vsel_ref, kdiag_ref, vdiag_ref,
                k0_ref, v0_ref, wo_ref, final_ref,
                out_ref, xpb_ref, sstage_ref, comm_ref, send_sems, recv_sems):
    g = pl.program_id(0)
    p = lax.axis_index("i")
    left = (p - 1 + N_DEV) % N_DEV
    right = (p + 1) % N_DEV

    @pl.when(g == 0)
    def _():
        barrier_sem = pltpu.get_barrier_semaphore()
        for nbr in (left, right):
            pl.semaphore_signal(
                barrier_sem, inc=1,
                device_id=(nbr,), device_id_type=pl.DeviceIdType.MESH,
            )
        pl.semaphore_wait(barrier_sem, 2)

    @pl.when(g < 3 * HEADS_PER_SHARD)
    def _():
        _compute_step(g, x_ref, wq_ref, ksel_ref, vsel_ref, kdiag_ref,
                      vdiag_ref, k0_ref, v0_ref, wo_ref, out_ref, xpb_ref)

    cs = g // HEADS_PER_SHARD - 1
    k = g - HEADS_PER_SHARD * (cs + 1)
    base = cs * ROWS

    def _desc(d, hop, src_ref, dst_ref, target):
        return pltpu.make_async_remote_copy(
            src_ref=src_ref, dst_ref=dst_ref,
            send_sem=send_sems.at[d, cs, hop], recv_sem=recv_sems.at[d, cs, hop],
            device_id=(target,), device_id_type=pl.DeviceIdType.MESH,
        )

    @pl.when((g >= HEADS_PER_SHARD) & (k >= 1) & (k <= N_HOPS))
    def _():
        ka = k - 1
        for d, tgt in ((0, right), (1, left)):
            desc = _desc(d, ka, sstage_ref.at[d, cs, 0], comm_ref.at[d, cs, ka], tgt)
            desc.wait_send()
            desc.wait_recv()

        @pl.when(ka <= N_DEV - 2)
        def _():
            j_cw = (p - ka - 1 + N_DEV) % N_DEV
            j_ccw = (p + ka + 1) % N_DEV
            out_ref[pl.ds(base + j_cw * CH, HF), :] += comm_ref[0, cs, ka].astype(jnp.float32)
            out_ref[pl.ds(base + j_ccw * CH + HF, HF), :] += comm_ref[1, cs, ka].astype(jnp.float32)

        @pl.when(ka > N_DEV - 2)
        def _():
            ta = ka - (N_DEV - 1)
            j_cw = (p - ta + N_DEV) % N_DEV
            j_ccw = (p + ta) % N_DEV
            out_ref[pl.ds(base + j_cw * CH, HF), :] = comm_ref[0, cs, ka].astype(jnp.float32)
            out_ref[pl.ds(base + j_ccw * CH + HF, HF), :] = comm_ref[1, cs, ka].astype(jnp.float32)

    @pl.when((g >= HEADS_PER_SHARD) & (k <= N_HOPS - 1))
    def _():
        @pl.when(k <= N_DEV - 2)
        def _():
            i_cw = (p - k + N_DEV) % N_DEV
            i_ccw = (p + k) % N_DEV
            sstage_ref[0, cs, k] = out_ref[pl.ds(base + i_cw * CH, HF), :].astype(jnp.bfloat16)
            sstage_ref[1, cs, k] = out_ref[pl.ds(base + i_ccw * CH + HF, HF), :].astype(jnp.bfloat16)
            for d, tgt in ((0, right), (1, left)):
                _desc(d, k, sstage_ref.at[d, cs, k], comm_ref.at[d, cs, k], tgt).start()

        @pl.when(k == N_DEV - 1)
        def _():
            own_cw = (p + 1) % N_DEV
            own_ccw = (p - 1 + N_DEV) % N_DEV
            sstage_ref[0, cs, k] = out_ref[pl.ds(base + own_cw * CH, HF), :].astype(jnp.bfloat16)
            sstage_ref[1, cs, k] = out_ref[pl.ds(base + own_ccw * CH + HF, HF), :].astype(jnp.bfloat16)
            for d, tgt in ((0, right), (1, left)):
                _desc(d, k, sstage_ref.at[d, cs, k], comm_ref.at[d, cs, k], tgt).start()

        @pl.when(k > N_DEV - 1)
        def _():
            for d, tgt in ((0, right), (1, left)):
                _desc(d, k, comm_ref.at[d, cs, k - 1], comm_ref.at[d, cs, k], tgt).start()

    first_copy = 2 * HEADS_PER_SHARD - 1
    @pl.when((g >= first_copy) & (lax.rem(g - first_copy, HEADS_PER_SHARD) == 0))
    def _():
        cc = (g - first_copy) // HEADS_PER_SHARD
        for i in range(NQB - 1):
            final_ref[pl.ds((3 * i + cc) * BLK, BLK), :] = (
                out_ref[pl.ds((cc * NQB + i) * BLK, BLK), :])

        @pl.when(cc <= 1)
        def _():
            final_ref[pl.ds((30 + cc) * BLK, BLK), :] = (
                out_ref[pl.ds((cc * NQB + NQB - 1) * BLK, BLK), :])


def _clamp_c(g):
    return jnp.minimum(g // HEADS_PER_SHARD, 2)


def _fused(x2d, wqb, kg, vg, wob):
    kg5 = kg.reshape(HEADS_PER_SHARD, 3, NQB, BLK, DH)
    vg5 = vg.reshape(HEADS_PER_SHARD, 3, NQB, BLK, DH)
    h_of = lambda g: lax.rem(g, HEADS_PER_SHARD)
    return pl.pallas_call(
        _fused_body,
        grid=(GRID,),
        in_specs=[
            pl.BlockSpec(memory_space=pltpu.VMEM),
            pl.BlockSpec((D_MODEL, DH), lambda g: (0, h_of(g))),
            pl.BlockSpec((1, 1, ROWS, DH), lambda g: (h_of(g), (3 - _clamp_c(g)) % 3, 0, 0)),
            pl.BlockSpec((1, 1, ROWS, DH), lambda g: (h_of(g), (3 - _clamp_c(g)) % 3, 0, 0)),
            pl.BlockSpec((1, 1, ROWS, DH), lambda g: (h_of(g), _clamp_c(g), 0, 0)),
            pl.BlockSpec((1, 1, ROWS, DH), lambda g: (h_of(g), _clamp_c(g), 0, 0)),
            pl.BlockSpec((1, 1, 1, BLK, DH), lambda g: (h_of(g), 0, 0, 0, 0)),
            pl.BlockSpec((1, 1, 1, BLK, DH), lambda g: (h_of(g), 0, 0, 0, 0)),
            pl.BlockSpec((1, DH, D_MODEL), lambda g: (h_of(g), 0, 0)),
        ],
        out_specs=pl.BlockSpec(memory_space=pltpu.VMEM),
        out_shape=jax.ShapeDtypeStruct((SQ, D_MODEL), jnp.float32),
        scratch_shapes=[
            pltpu.VMEM((RTOT, D_MODEL), jnp.float32),
            pltpu.VMEM((ROWS, D_MODEL), jnp.bfloat16),
            pltpu.VMEM((2, 3, N_DEV, HF, D_MODEL), jnp.bfloat16),
            pltpu.VMEM((2, 3, N_HOPS, HF, D_MODEL), jnp.bfloat16),
            pltpu.SemaphoreType.DMA((2, 3, N_HOPS)),
            pltpu.SemaphoreType.DMA((2, 3, N_HOPS)),
        ],
        compiler_params=pltpu.CompilerParams(collective_id=0),
    )(x2d, wqb, kg, vg, kg, vg, kg5, vg5, wob)


def kernel(x, Wq, K_ext, V_ext, Wo):
    p = lax.axis_index("i")

    wqb = Wq.astype(jnp.bfloat16)

    k = lax.dynamic_slice_in_dim(K_ext[0], p * HEADS_PER_SHARD, HEADS_PER_SHARD, axis=1)
    v = lax.dynamic_slice_in_dim(V_ext[0], p * HEADS_PER_SHARD, HEADS_PER_SHARD, axis=1)
    kb = jnp.transpose(k, (1, 0, 2)).astype(jnp.bfloat16)
    vb = jnp.transpose(v, (1, 0, 2)).astype(jnp.bfloat16)
    kg = _group_blocks(kb.reshape(HEADS_PER_SHARD, 32, BLK, DH))
    vg = _group_blocks(vb.reshape(HEADS_PER_SHARD, 32, BLK, DH))

    wob = Wo.reshape(HEADS_PER_SHARD, DH, D_MODEL).astype(jnp.bfloat16)

    total = _fused(x[0], wqb, kg, vg, wob)
    return total[None]
